# baseline (speedup 1.0000x reference)
import functools

import numpy as np
import jax
import jax.numpy as jnp
from jax.experimental import pallas as pl
from jax.experimental.pallas import tpu as pltpu

L = 400
C = 256
UP = 1600
TILE = 160
BAND = 56
LT = 100
LTP = 128
NT = UP // TILE


def _pool1d_matrix(l_in, l_out):
    w = np.zeros((l_in, l_out), dtype=np.float32)
    for i in range(l_out):
        s = (i * l_in) // l_out
        e = -(((-(i + 1)) * l_in) // l_out)
        w[s:e, i] = 1.0 / (e - s)
    return w


def _band_offset(t):
    return min((((TILE * t * (L - 1)) // (UP - 1)) // 8) * 8, L - BAND)


def _bilinear_bands():
    ys = np.linspace(0.0, L - 1.0, UP, dtype=np.float32)
    y0 = np.floor(ys).astype(np.int64)
    y1 = np.minimum(y0 + 1, L - 1)
    wy = ys - y0.astype(np.float32)
    a = np.zeros((UP, L), dtype=np.float32)
    for i in range(UP):
        a[i, y0[i]] += 1.0 - wy[i]
        a[i, y1[i]] += wy[i]
    s = np.zeros((UP, BAND), dtype=np.float32)
    bp = np.zeros((NT * BAND, TILE), dtype=np.float32)
    for t in range(NT):
        off = _band_offset(t)
        s[TILE * t:TILE * (t + 1), :] = a[TILE * t:TILE * (t + 1),
                                          off:off + BAND]
        bp[BAND * t:BAND * (t + 1), :] = a[TILE * t:TILE * (t + 1),
                                           off:off + BAND].T
    return s, bp


def _spatial_pool_matrix():
    p = np.zeros((LTP, L), dtype=np.float32)
    for j in range(LT):
        rr, cc = divmod(j, 10)
        for dr in range(2):
            for dc in range(2):
                p[j, (2 * rr + dr) * 20 + (2 * cc + dc)] = 0.25
    return p


@functools.cache
def _consts():
    s, bp = _bilinear_bands()
    wa = _pool1d_matrix(C + 1, C)
    wb = _pool1d_matrix(2 * C, C)
    wal = np.zeros((8, C), dtype=np.float32)
    wal[0] = wa[C]
    return (s, bp, wa[:C].copy(), wal, wb[:C].copy(), wb[C:].copy(),
            _spatial_pool_matrix())


def _dot(x, y):
    return jax.lax.dot_general(x, y, (((1,), (0,)), ((), ())),
                               preferred_element_type=jnp.float32)


def _fused_kernel(x0_ref, x1_ref, t32_ref, s_ref, bp_ref, p_ref, wat_ref,
                  wal_ref, wbt_ref, wbb_ref,
                  conf_f_ref, f0_ref, f1_ref, t32p_ref, tmp_ref):
    g = pl.program_id(0)
    scale = 1.0 / (C ** 0.5)

    @pl.when(g == 0)
    def _phase0():
        conf = jax.lax.dot_general(x0_ref[...], x1_ref[...],
                                   (((1,), (1,)), ((), ())),
                                   preferred_element_type=jnp.float32) * scale
        for t in range(NT):
            off = _band_offset(t)
            tmp_ref[:, TILE * t:TILE * (t + 1)] = _dot(
                conf[:, off:off + BAND],
                bp_ref[BAND * t:BAND * (t + 1), :])

    @pl.when(g == 1)
    def _epilogue():
        x0 = x0_ref[...]
        x1 = x1_ref[...]
        conf = jax.lax.dot_general(x0, x1, (((1,), (1,)), ((), ())),
                                   preferred_element_type=jnp.float32) * scale
        confT = jax.lax.dot_general(x1, x0, (((1,), (1,)), ((), ())),
                                    preferred_element_type=jnp.float32) * scale

        def dual_softmax(cm):
            e2 = jnp.exp(cm - jnp.max(cm, axis=1, keepdims=True))
            sm2 = e2 / jnp.sum(e2, axis=1, keepdims=True)
            e1 = jnp.exp(cm - jnp.max(cm, axis=0, keepdims=True))
            sm1 = e1 / jnp.sum(e1, axis=0, keepdims=True)
            return sm1 * sm2

        iota_s = jax.lax.broadcasted_iota(jnp.int32, (L, L), 1)

        def row_argmax(cm):
            m = jnp.max(cm, axis=1, keepdims=True)
            return jnp.min(jnp.where(cm == m, iota_s, jnp.int32(1 << 30)),
                           axis=1, keepdims=True)

        confm = dual_softmax(conf)
        confmT = dual_softmax(confT)
        idx = row_argmax(confm)
        idy = row_argmax(confmT)

        onehot = (iota_s == idx).astype(jnp.float32)
        gsel = _dot(onehot, x1)
        ft100 = _dot(p_ref[...], (x0 + gsel) * 0.5)

        t32p_ref[...] = (_dot(t32_ref[...], wbt_ref[...]) +
                         _dot(ft100, wbb_ref[...]))
        wal = wal_ref[0:1, :]
        idxf = idx.astype(jnp.float32).astype(jnp.bfloat16).astype(jnp.float32)
        idyf = idy.astype(jnp.float32).astype(jnp.bfloat16).astype(jnp.float32)
        f0_ref[0] = _dot(x0, wat_ref[...]) + idxf * wal
        f1_ref[0] = _dot(x1, wat_ref[...]) + idyf * wal

    start = jnp.minimum(((TILE * g * (L - 1)) // (UP - 1)) // 8 * 8, L - BAND)
    conf_f_ref[0, 0] = _dot(s_ref[...], tmp_ref[pl.ds(start, BAND), :])


def kernel(feats16_0, feats16_1, feats8_0, feats8_1, feats4_0, feats4_1,
           t64, t32, t16):
    del feats8_0, feats8_1, feats4_0, feats4_1, t64, t16
    s, bp, wat, wal, wbt, wbb, p = (jnp.asarray(c) for c in _consts())
    x0 = feats16_0[0]
    x1 = feats16_1[0]
    t32pad = jnp.pad(t32[0], ((0, LTP - LT), (0, 0)))

    conf_f, f0, f1, t32p = pl.pallas_call(
        _fused_kernel,
        grid=(NT,),
        in_specs=[
            pl.BlockSpec((L, C), lambda g: (0, 0)),
            pl.BlockSpec((L, C), lambda g: (0, 0)),
            pl.BlockSpec((LTP, C), lambda g: (0, 0)),
            pl.BlockSpec((TILE, BAND), lambda g: (g, 0)),
            pl.BlockSpec((NT * BAND, TILE), lambda g: (0, 0)),
            pl.BlockSpec((LTP, L), lambda g: (0, 0)),
            pl.BlockSpec((C, C), lambda g: (0, 0)),
            pl.BlockSpec((8, C), lambda g: (0, 0)),
            pl.BlockSpec((C, C), lambda g: (0, 0)),
            pl.BlockSpec((C, C), lambda g: (0, 0)),
        ],
        out_specs=[
            pl.BlockSpec((1, 1, TILE, UP), lambda g: (0, 0, g, 0)),
            pl.BlockSpec((1, L, C), lambda g: (0, 0, 0)),
            pl.BlockSpec((1, L, C), lambda g: (0, 0, 0)),
            pl.BlockSpec((LTP, C), lambda g: (0, 0)),
        ],
        out_shape=[
            jax.ShapeDtypeStruct((1, 1, UP, UP), jnp.float32),
            jax.ShapeDtypeStruct((1, L, C), jnp.float32),
            jax.ShapeDtypeStruct((1, L, C), jnp.float32),
            jax.ShapeDtypeStruct((LTP, C), jnp.float32),
        ],
        scratch_shapes=[pltpu.VMEM((L, UP), jnp.float32)],
    )(x0, x1, t32pad, s, bp, p, wat, wal, wbt, wbb)

    return (conf_f, f0, f1, t32p[:LT, None, :])


# speedup vs baseline: 1.1122x; 1.1122x over previous
import jax
import jax.numpy as jnp
from jax.experimental import pallas as pl
from jax.experimental.pallas import tpu as pltpu

L = 400
C = 256
UP = 1600
TILE = 160
BAND = 56
LT = 100
LTP = 128
NT = UP // TILE
DELTA = float((L - 1) / (UP - 1))


def _band_offset(t):
    return min((((TILE * t * (L - 1)) // (UP - 1)) // 8) * 8, L - BAND)


def _dot(x, y):
    return jax.lax.dot_general(x, y, (((1,), (0,)), ((), ())),
                               preferred_element_type=jnp.float32)


def _interp_band(shape, axis, base, off):
    out_ax, band_ax = (1 - axis, axis)
    ci = jax.lax.broadcasted_iota(jnp.int32, shape, out_ax) + base
    jj = jax.lax.broadcasted_iota(jnp.int32, shape, band_ax)
    ys = ci.astype(jnp.float32) * DELTA
    y0f = jnp.floor(ys)
    w = ys - y0f
    j0 = y0f.astype(jnp.int32) - off
    m0 = (jj == j0).astype(jnp.float32)
    m1 = (jj == j0 + 1).astype(jnp.float32)
    return m0 * (1.0 - w) + m1 * w


def _fused_kernel(x0_ref, x1_ref, t32_ref,
                  conf_f_ref, f0_ref, f1_ref, t32p_ref,
                  tmp_ref, conf_ref, confT_ref):
    g = pl.program_id(0)
    scale = 1.0 / (C ** 0.5)

    @pl.when(g == 0)
    def _phase0():
        x0 = x0_ref[...]
        x1 = x1_ref[...]
        conf = jax.lax.dot_general(x0, x1, (((1,), (1,)), ((), ())),
                                   preferred_element_type=jnp.float32) * scale
        confT = jax.lax.dot_general(x1, x0, (((1,), (1,)), ((), ())),
                                    preferred_element_type=jnp.float32) * scale
        conf_ref[...] = conf
        confT_ref[...] = confT
        for t in range(NT):
            off = _band_offset(t)
            bp = _interp_band((BAND, TILE), 0, TILE * t, off)
            tmp_ref[:, TILE * t:TILE * (t + 1)] = _dot(
                conf[:, off:off + BAND], bp)

    @pl.when(g == 1)
    def _epilogue():
        x0 = x0_ref[...]
        x1 = x1_ref[...]
        conf = conf_ref[...]
        confT = confT_ref[...]

        def dual_softmax(cm):
            e2 = jnp.exp(cm - jnp.max(cm, axis=1, keepdims=True))
            sm2 = e2 / jnp.sum(e2, axis=1, keepdims=True)
            e1 = jnp.exp(cm - jnp.max(cm, axis=0, keepdims=True))
            sm1 = e1 / jnp.sum(e1, axis=0, keepdims=True)
            return sm1 * sm2

        iota_s = jax.lax.broadcasted_iota(jnp.int32, (L, L), 1)

        def row_argmax(cm):
            m = jnp.max(cm, axis=1, keepdims=True)
            return jnp.min(jnp.where(cm == m, iota_s, jnp.int32(1 << 30)),
                           axis=1, keepdims=True)

        idx = row_argmax(dual_softmax(conf))
        idy = row_argmax(dual_softmax(confT))

        onehot = (iota_s == idx).astype(jnp.float32)
        gsel = _dot(onehot, x1)

        pj = jax.lax.broadcasted_iota(jnp.int32, (LTP, L), 0)
        plq = jax.lax.broadcasted_iota(jnp.int32, (LTP, L), 1)
        pool = jnp.where((plq // 40 == pj // 10) & ((plq % 20) // 2 == pj % 10)
                         & (pj < LT), 0.25, 0.0)
        ft100 = _dot(pool, (x0 + gsel) * 0.5)

        ik = jax.lax.broadcasted_iota(jnp.int32, (C, C), 0)
        il = jax.lax.broadcasted_iota(jnp.int32, (C, C), 1)
        wbt = jnp.where(ik // 2 == il, 0.5, 0.0)
        wbb = jnp.where(ik // 2 + 128 == il, 0.5, 0.0)
        t32p_ref[...] = _dot(t32_ref[...], wbt) + _dot(ft100, wbb)

        idxf = idx.astype(jnp.float32).astype(jnp.bfloat16).astype(jnp.float32)
        idyf = idy.astype(jnp.float32).astype(jnp.bfloat16).astype(jnp.float32)
        lane = jax.lax.broadcasted_iota(jnp.int32, (L, C), 1)
        last = lane == C - 1

        def shift_avg(x, idf):
            nxt = jnp.concatenate([x[:, 1:], x[:, :1]], axis=1)
            return 0.5 * x + 0.5 * jnp.where(last, idf, nxt)

        f0_ref[0] = shift_avg(x0, idxf)
        f1_ref[0] = shift_avg(x1, idyf)

    start = jnp.minimum(((TILE * g * (L - 1)) // (UP - 1)) // 8 * 8, L - BAND)
    s = _interp_band((TILE, BAND), 1, TILE * g, start)
    conf_f_ref[0, 0] = _dot(s, tmp_ref[pl.ds(start, BAND), :])


def kernel(feats16_0, feats16_1, feats8_0, feats8_1, feats4_0, feats4_1,
           t64, t32, t16):
    del feats8_0, feats8_1, feats4_0, feats4_1, t64, t16
    x0 = feats16_0[0]
    x1 = feats16_1[0]
    t32pad = jnp.pad(t32[0], ((0, LTP - LT), (0, 0)))

    conf_f, f0, f1, t32p = pl.pallas_call(
        _fused_kernel,
        grid=(NT,),
        in_specs=[
            pl.BlockSpec((L, C), lambda g: (0, 0)),
            pl.BlockSpec((L, C), lambda g: (0, 0)),
            pl.BlockSpec((LTP, C), lambda g: (0, 0)),
        ],
        out_specs=[
            pl.BlockSpec((1, 1, TILE, UP), lambda g: (0, 0, g, 0)),
            pl.BlockSpec((1, L, C), lambda g: (0, 0, 0)),
            pl.BlockSpec((1, L, C), lambda g: (0, 0, 0)),
            pl.BlockSpec((LTP, C), lambda g: (0, 0)),
        ],
        out_shape=[
            jax.ShapeDtypeStruct((1, 1, UP, UP), jnp.float32),
            jax.ShapeDtypeStruct((1, L, C), jnp.float32),
            jax.ShapeDtypeStruct((1, L, C), jnp.float32),
            jax.ShapeDtypeStruct((LTP, C), jnp.float32),
        ],
        scratch_shapes=[pltpu.VMEM((L, UP), jnp.float32),
                        pltpu.VMEM((L, L), jnp.float32),
                        pltpu.VMEM((L, L), jnp.float32)],
    )(x0, x1, t32pad)

    return (conf_f, f0, f1, t32p[:LT, None, :])
